# baseline (speedup 1.0000x reference)
import functools
import math

import jax
import jax.numpy as jnp
from jax import lax
from jax.experimental import pallas as pl
from jax.experimental.pallas import tpu as pltpu
from jax.experimental.pallas import tpu_sc as plsc

VOCAB = 1000000
EMB = 64
PACKW = 128
SCALE = math.sqrt(EMB)

NC = 2
NS = 16
NW = NC * NS

BATCH = 4096
TSTEP = 200
BPW = BATCH // NW
TPAD = 208
TOKW = 256
NCH = TPAD // 16
MP = 17
LANES = 16
VPR = EMB // LANES
RING = 2
NGRP = BPW // RING


def _emb_kernel_body(table_hbm, tok_hbm, out_hbm,
                     idxraw0, idxraw1, rowbuf0, rowbuf1, maskbuf0, maskbuf1,
                     slab0, slab1, outbuf0, outbuf1, isem, gsem, osem):
    idxraw = [idxraw0, idxraw1]
    rowbuf = [rowbuf0, rowbuf1]
    maskbuf = [maskbuf0, maskbuf1]
    slab = [slab0, slab1]
    outbuf = [outbuf0, outbuf1]

    c = lax.axis_index("c")
    s = lax.axis_index("s")
    wid = s * NC + c
    b0 = wid * BPW

    lanes = lax.iota(jnp.int32, LANES)
    head8 = lanes < 8

    def fire_idx(b, r):
        pltpu.async_copy(tok_hbm.at[b0 + b], idxraw[r], isem.at[r])

    def wait_idx(b, r):
        pltpu.make_async_copy(tok_hbm.at[b0 + b], idxraw[r],
                              isem.at[r]).wait()

    def idx_phase(r):
        for i in range(NCH):
            v = idxraw[r][pl.ds(16 * i, LANES)]
            if i == NCH - 1:
                v = jnp.where(head8, v, 0)
            rowbuf[r][pl.ds(16 * i, LANES)] = lax.shift_right_logical(v, 1)
            p = lax.bitwise_and(v, 1)
            tbase = (lanes + 16 * i) * MP
            for col in range(LANES):
                plsc.store_scatter(maskbuf[r], [tbase + col], p)

    def fire_gather(r):
        pltpu.async_copy(table_hbm.at[rowbuf[r].at[pl.ds(0, 128)]],
                         slab[r].at[pl.ds(0, 128)], gsem.at[r])
        pltpu.async_copy(table_hbm.at[rowbuf[r].at[pl.ds(128, TPAD - 128)]],
                         slab[r].at[pl.ds(128, TPAD - 128)], gsem.at[r])

    def wait_gather(r):
        pltpu.make_async_copy(table_hbm.at[rowbuf[r].at[pl.ds(0, 128)]],
                              slab[r].at[pl.ds(0, 128)], gsem.at[r]).wait()
        pltpu.make_async_copy(table_hbm.at[rowbuf[r].at[pl.ds(128,
                                                              TPAD - 128)]],
                              slab[r].at[pl.ds(128, TPAD - 128)],
                              gsem.at[r]).wait()

    def fire_store(b, r):
        pltpu.async_copy(outbuf[r].at[pl.ds(0, TSTEP)],
                         out_hbm.at[b0 + b], osem.at[r])

    def wait_store(b, r):
        pltpu.make_async_copy(outbuf[r].at[pl.ds(0, TSTEP)],
                              out_hbm.at[b0 + b], osem.at[r]).wait()

    def select_phase(r):
        def tok_body(t, kc):
            m = maskbuf[r][pl.ds(t * MP, LANES)] != 0
            for cc in range(VPR):
                lo = slab[r][t, pl.ds(16 * cc, LANES)]
                hi = slab[r][t, pl.ds(EMB + 16 * cc, LANES)]
                outbuf[r][t, pl.ds(16 * cc, LANES)] = (
                    jnp.where(m, hi, lo) * SCALE)
            return kc

        lax.fori_loop(0, TSTEP, tok_body, 0)

    fire_idx(0, 0)
    wait_idx(0, 0)
    idx_phase(0)
    fire_gather(0)
    fire_idx(1, 1)

    def group(g, carry):
        for r in range(RING):
            b = g * RING + r

            wait_gather(r)

            @pl.when(b >= RING)
            def _():
                wait_store(b - RING, r)

            select_phase(r)
            fire_store(b, r)

            nr = (r + 1) % RING

            @pl.when(b + 1 < BPW)
            def _():
                wait_idx(b + 1, nr)
                idx_phase(nr)
                fire_gather(nr)

            @pl.when(b + 2 < BPW)
            def _():
                fire_idx(b + 2, r)

        return carry

    lax.fori_loop(0, NGRP, group, 0)

    for r in range(RING):
        wait_store(BPW - RING + r, (BPW - RING + r) % RING)


@jax.jit
def _emb_lookup(table_packed, tokens):
    mesh = plsc.VectorSubcoreMesh(core_axis_name="c", subcore_axis_name="s")
    fn = pl.kernel(
        _emb_kernel_body,
        mesh=mesh,
        out_type=jax.ShapeDtypeStruct((BATCH, TSTEP, EMB), jnp.float32),
        scratch_types=[
            pltpu.VMEM((TOKW,), jnp.int32),
            pltpu.VMEM((TOKW,), jnp.int32),
            pltpu.VMEM((TPAD,), jnp.int32),
            pltpu.VMEM((TPAD,), jnp.int32),
            pltpu.VMEM((TPAD * MP,), jnp.int32),
            pltpu.VMEM((TPAD * MP,), jnp.int32),
            pltpu.VMEM((TPAD, PACKW), jnp.float32),
            pltpu.VMEM((TPAD, PACKW), jnp.float32),
            pltpu.VMEM((TPAD, EMB), jnp.float32),
            pltpu.VMEM((TPAD, EMB), jnp.float32),
            pltpu.SemaphoreType.DMA((RING,)),
            pltpu.SemaphoreType.DMA((RING,)),
            pltpu.SemaphoreType.DMA((RING,)),
        ],
        compiler_params=pltpu.CompilerParams(use_tc_tiling_on_sc=True,
                                             needs_layout_passes=False),
    )
    return fn(table_packed, tokens)


def kernel(tokens, table):
    packed = table.reshape(VOCAB // 2, PACKW)
    tok_p = jnp.pad(tokens.astype(jnp.int32), ((0, 0), (0, TOKW - TSTEP)))
    return _emb_lookup(packed, tok_p)


# speedup vs baseline: 1.9189x; 1.9189x over previous
import functools
import math

import jax
import jax.numpy as jnp
from jax import lax
from jax.experimental import pallas as pl
from jax.experimental.pallas import tpu as pltpu
from jax.experimental.pallas import tpu_sc as plsc

VOCAB = 1000000
EMB = 64
PADW = 128
SCALE = math.sqrt(EMB)

NC = 2
NS = 16
NW = NC * NS

B_TOTAL = 4096 * 200
BPW = B_TOTAL // NW
CHUNK = 128
NCHUNK = BPW // CHUNK
LANES = 16
VPR = EMB // LANES
RING = 3


def _emb_kernel_body(table_hbm, idx_hbm, out_hbm, idx_v,
                     gbuf0, gbuf1, gbuf2, obuf0, obuf1, obuf2, gsem, osem):
    gbuf = [gbuf0, gbuf1, gbuf2]
    obuf = [obuf0, obuf1, obuf2]

    c = lax.axis_index("c")
    s = lax.axis_index("s")
    wid = s * NC + c

    pltpu.sync_copy(idx_hbm.at[wid], idx_v)

    def fire_gather(j, r):
        pltpu.async_copy(table_hbm.at[idx_v.at[pl.ds(j * CHUNK, CHUNK)]],
                         gbuf[r], gsem.at[r])

    def wait_gather(j, r):
        pltpu.make_async_copy(table_hbm.at[idx_v.at[pl.ds(j * CHUNK, CHUNK)]],
                              gbuf[r], gsem.at[r]).wait()

    def fire_store(j, r):
        pltpu.async_copy(obuf[r], out_hbm.at[wid, pl.ds(j * CHUNK, CHUNK)],
                         osem.at[r])

    def wait_store(j, r):
        pltpu.make_async_copy(obuf[r],
                              out_hbm.at[wid, pl.ds(j * CHUNK, CHUNK)],
                              osem.at[r]).wait()

    for r in range(RING):
        fire_gather(r, r)

    def group(g, carry):
        for r in range(RING):
            j = g * RING + r

            wait_gather(j, r)

            @pl.when(j >= RING)
            def _():
                wait_store(j - RING, r)

            def row_body(k, kc):
                for cc in range(VPR):
                    sl = pl.ds(16 * cc, LANES)
                    obuf[r][k, sl] = gbuf[r][k, sl] * SCALE
                return kc

            lax.fori_loop(0, CHUNK, row_body, 0)

            fire_store(j, r)

            @pl.when(j + RING < NCHUNK)
            def _():
                fire_gather(j + RING, r)

        return carry

    lax.fori_loop(0, NCHUNK // RING, group, 0)

    for j in range((NCHUNK // RING) * RING, NCHUNK):
        r = j % RING
        wait_gather(j, r)
        wait_store(j - RING, r)

        def row_body(k, kc):
            for cc in range(VPR):
                sl = pl.ds(16 * cc, LANES)
                obuf[r][k, sl] = gbuf[r][k, sl] * SCALE
            return kc

        lax.fori_loop(0, CHUNK, row_body, 0)
        fire_store(j, r)

    for j in range(NCHUNK - RING, NCHUNK):
        wait_store(j, j % RING)


@jax.jit
def _emb_lookup(table_padded, idx):
    mesh = plsc.VectorSubcoreMesh(core_axis_name="c", subcore_axis_name="s")
    fn = pl.kernel(
        _emb_kernel_body,
        mesh=mesh,
        out_type=jax.ShapeDtypeStruct((NW, BPW, EMB), jnp.float32),
        scratch_types=[
            pltpu.VMEM((BPW,), jnp.int32),
            pltpu.VMEM((CHUNK, PADW), jnp.float32),
            pltpu.VMEM((CHUNK, PADW), jnp.float32),
            pltpu.VMEM((CHUNK, PADW), jnp.float32),
            pltpu.VMEM((CHUNK, EMB), jnp.float32),
            pltpu.VMEM((CHUNK, EMB), jnp.float32),
            pltpu.VMEM((CHUNK, EMB), jnp.float32),
            pltpu.SemaphoreType.DMA((RING,)),
            pltpu.SemaphoreType.DMA((RING,)),
        ],
        compiler_params=pltpu.CompilerParams(use_tc_tiling_on_sc=False,
                                             needs_layout_passes=False),
    )
    return fn(table_padded, idx)


def kernel(tokens, table):
    padded = jnp.pad(table, ((0, 0), (0, PADW - EMB)))
    idx = tokens.astype(jnp.int32).reshape(NW, BPW)
    out = _emb_lookup(padded, idx)
    return out.reshape(4096, 200, EMB)


# speedup vs baseline: 1.9265x; 1.0039x over previous
import functools
import math

import jax
import jax.numpy as jnp
from jax import lax
from jax.experimental import pallas as pl
from jax.experimental.pallas import tpu as pltpu
from jax.experimental.pallas import tpu_sc as plsc

VOCAB = 1000000
EMB = 64
PADW = 128
SCALE = math.sqrt(EMB)

NC = 2
NS = 16
NW = NC * NS

B_TOTAL = 4096 * 200
BPW = B_TOTAL // NW
CHUNK = 128
NCHUNK = BPW // CHUNK
LANES = 16
VPR = EMB // LANES
RING = 4


def _emb_kernel_body(table_hbm, idx_hbm, out_hbm, idx_v,
                     gbuf0, gbuf1, gbuf2, gbuf3, obuf0, obuf1, obuf2, obuf3,
                     gsem, osem):
    gbuf = [gbuf0, gbuf1, gbuf2, gbuf3]
    obuf = [obuf0, obuf1, obuf2, obuf3]

    c = lax.axis_index("c")
    s = lax.axis_index("s")
    wid = s * NC + c

    pltpu.sync_copy(idx_hbm.at[wid], idx_v)

    def fire_gather(j, r):
        pltpu.async_copy(table_hbm.at[idx_v.at[pl.ds(j * CHUNK, CHUNK)]],
                         gbuf[r], gsem.at[r])

    def wait_gather(j, r):
        pltpu.make_async_copy(table_hbm.at[idx_v.at[pl.ds(j * CHUNK, CHUNK)]],
                              gbuf[r], gsem.at[r]).wait()

    def fire_store(j, r):
        pltpu.async_copy(obuf[r], out_hbm.at[wid, pl.ds(j * CHUNK, CHUNK)],
                         osem.at[r])

    def wait_store(j, r):
        pltpu.make_async_copy(obuf[r],
                              out_hbm.at[wid, pl.ds(j * CHUNK, CHUNK)],
                              osem.at[r]).wait()

    for r in range(RING):
        fire_gather(r, r)

    def group(g, carry):
        for r in range(RING):
            j = g * RING + r

            wait_gather(j, r)

            @pl.when(j >= RING)
            def _():
                wait_store(j - RING, r)

            def row_body(k, kc):
                for cc in range(VPR):
                    sl = pl.ds(16 * cc, LANES)
                    obuf[r][k, sl] = gbuf[r][k, sl] * SCALE
                return kc

            lax.fori_loop(0, CHUNK, row_body, 0)

            fire_store(j, r)

            @pl.when(j + RING < NCHUNK)
            def _():
                fire_gather(j + RING, r)

        return carry

    lax.fori_loop(0, NCHUNK // RING, group, 0)

    for j in range((NCHUNK // RING) * RING, NCHUNK):
        r = j % RING
        wait_gather(j, r)
        wait_store(j - RING, r)

        def row_body(k, kc):
            for cc in range(VPR):
                sl = pl.ds(16 * cc, LANES)
                obuf[r][k, sl] = gbuf[r][k, sl] * SCALE
            return kc

        lax.fori_loop(0, CHUNK, row_body, 0)
        fire_store(j, r)

    for j in range(NCHUNK - RING, NCHUNK):
        wait_store(j, j % RING)


@jax.jit
def _emb_lookup(table_padded, idx):
    mesh = plsc.VectorSubcoreMesh(core_axis_name="c", subcore_axis_name="s")
    fn = pl.kernel(
        _emb_kernel_body,
        mesh=mesh,
        out_type=jax.ShapeDtypeStruct((NW, BPW, EMB), jnp.float32),
        scratch_types=[
            pltpu.VMEM((BPW,), jnp.int32),
            pltpu.VMEM((CHUNK, PADW), jnp.float32),
            pltpu.VMEM((CHUNK, PADW), jnp.float32),
            pltpu.VMEM((CHUNK, PADW), jnp.float32),
            pltpu.VMEM((CHUNK, PADW), jnp.float32),
            pltpu.VMEM((CHUNK, EMB), jnp.float32),
            pltpu.VMEM((CHUNK, EMB), jnp.float32),
            pltpu.VMEM((CHUNK, EMB), jnp.float32),
            pltpu.VMEM((CHUNK, EMB), jnp.float32),
            pltpu.SemaphoreType.DMA((RING,)),
            pltpu.SemaphoreType.DMA((RING,)),
        ],
        compiler_params=pltpu.CompilerParams(use_tc_tiling_on_sc=False,
                                             needs_layout_passes=False),
    )
    return fn(table_padded, idx)


def kernel(tokens, table):
    padded = jnp.pad(table, ((0, 0), (0, PADW - EMB)))
    idx = tokens.astype(jnp.int32).reshape(NW, BPW)
    out = _emb_lookup(padded, idx)
    return out.reshape(4096, 200, EMB)


# speedup vs baseline: 2.3467x; 1.2181x over previous
import functools
import math

import jax
import jax.numpy as jnp
from jax import lax
from jax.experimental import pallas as pl
from jax.experimental.pallas import tpu as pltpu
from jax.experimental.pallas import tpu_sc as plsc

VOCAB = 1000000
EMB = 64
SCALE = math.sqrt(EMB)

NC = 2
NS = 16
NW = NC * NS

B_TOTAL = 4096 * 200
BPW = B_TOTAL // NW
CHUNK = 128
NCHUNK = BPW // CHUNK
LANES = 16
VPR = EMB // LANES

NBUF = 4
NGROUP = NCHUNK // NBUF


def _emb_kernel_body(table_hbm, idx_hbm, out_hbm, idx_v, in_bufs, out_bufs,
                     gsem, ssem):
    c = lax.axis_index("c")
    s = lax.axis_index("s")
    wid = s * NC + c

    pltpu.sync_copy(idx_hbm.at[wid], idx_v)

    for b in range(NBUF):
        pltpu.async_copy(table_hbm.at[idx_v.at[b]], in_bufs.at[b], gsem.at[b])

    def group_body(g, carry):
        for b in range(NBUF):
            j = g * NBUF + b

            @pl.when(g > 0)
            def _():
                pltpu.make_async_copy(out_bufs.at[b], out_hbm.at[wid, j],
                                      ssem.at[b]).wait()

            pltpu.make_async_copy(table_hbm.at[idx_v.at[j]], in_bufs.at[b],
                                  gsem.at[b]).wait()

            def row_body(r, rc):
                for cc in range(VPR):
                    sl = pl.ds(cc * LANES, LANES)
                    out_bufs[b, r, sl] = in_bufs[b, r, sl] * SCALE
                return rc

            lax.fori_loop(0, CHUNK, row_body, 0)

            pltpu.async_copy(out_bufs.at[b], out_hbm.at[wid, j], ssem.at[b])

            @pl.when(j + NBUF < NCHUNK)
            def _():
                pltpu.async_copy(table_hbm.at[idx_v.at[j + NBUF]],
                                 in_bufs.at[b], gsem.at[b])

        return carry

    lax.fori_loop(0, NGROUP, group_body, 0)

    for b in range(NBUF):
        pltpu.make_async_copy(out_bufs.at[b], out_hbm.at[wid, NCHUNK - NBUF + b],
                              ssem.at[b]).wait()


@jax.jit
def _emb_lookup(table, idx):
    mesh = plsc.VectorSubcoreMesh(core_axis_name="c", subcore_axis_name="s")
    fn = pl.kernel(
        _emb_kernel_body,
        mesh=mesh,
        out_type=jax.ShapeDtypeStruct((NW, NCHUNK, CHUNK, EMB), jnp.float32),
        scratch_types=[
            pltpu.VMEM((NCHUNK, CHUNK), jnp.int32),
            pltpu.VMEM((NBUF, CHUNK, EMB), jnp.float32),
            pltpu.VMEM((NBUF, CHUNK, EMB), jnp.float32),
            pltpu.SemaphoreType.DMA((NBUF,)),
            pltpu.SemaphoreType.DMA((NBUF,)),
        ],
        compiler_params=pltpu.CompilerParams(use_tc_tiling_on_sc=False),
    )
    return fn(table, idx)


def kernel(tokens, table):
    idx = tokens.reshape(NW, NCHUNK, CHUNK).astype(jnp.int32)
    out = _emb_lookup(table, idx)
    return out.reshape(4096, 200, EMB)


# speedup vs baseline: 2.3516x; 1.0021x over previous
import functools
import math

import jax
import jax.numpy as jnp
from jax import lax
from jax.experimental import pallas as pl
from jax.experimental.pallas import tpu as pltpu
from jax.experimental.pallas import tpu_sc as plsc

VOCAB = 1000000
EMB = 64
SCALE = math.sqrt(EMB)

NC = 2
NS = 16
NW = NC * NS

BATCH = 4096
TSTEP = 200
BPW = BATCH // NW
G1 = 128
G2 = TSTEP - G1
LANES = 16
VPR = EMB // LANES
RING = 3


def _emb_kernel_body(table_hbm, tok_hbm, out_hbm, idx_v,
                     gbuf0, gbuf1, gbuf2, obuf0, obuf1, obuf2, gsem, osem):
    gbuf = [gbuf0, gbuf1, gbuf2]
    obuf = [obuf0, obuf1, obuf2]

    c = lax.axis_index("c")
    s = lax.axis_index("s")
    wid = s * NC + c
    b0 = wid * BPW

    pltpu.sync_copy(tok_hbm.at[pl.ds(b0, BPW)], idx_v)

    def fire_gather(b, r):
        pltpu.async_copy(table_hbm.at[idx_v.at[b, pl.ds(0, G1)]],
                         gbuf[r].at[pl.ds(0, G1)], gsem.at[r])
        pltpu.async_copy(table_hbm.at[idx_v.at[b, pl.ds(G1, G2)]],
                         gbuf[r].at[pl.ds(G1, G2)], gsem.at[r])

    def wait_gather(b, r):
        pltpu.make_async_copy(table_hbm.at[idx_v.at[b, pl.ds(0, G1)]],
                              gbuf[r].at[pl.ds(0, G1)], gsem.at[r]).wait()
        pltpu.make_async_copy(table_hbm.at[idx_v.at[b, pl.ds(G1, G2)]],
                              gbuf[r].at[pl.ds(G1, G2)], gsem.at[r]).wait()

    def fire_store(b, r):
        pltpu.async_copy(obuf[r], out_hbm.at[b0 + b], osem.at[r])

    def wait_store(b, r):
        pltpu.make_async_copy(obuf[r], out_hbm.at[b0 + b], osem.at[r]).wait()

    for r in range(RING):
        fire_gather(r, r)

    def group(g, carry):
        for r in range(RING):
            b = g * RING + r

            wait_gather(b, r)

            @pl.when(b >= RING)
            def _():
                wait_store(b - RING, r)

            def row_body(t, tc_):
                for cc in range(VPR):
                    sl = pl.ds(16 * cc, LANES)
                    obuf[r][t, sl] = gbuf[r][t, sl] * SCALE
                return tc_

            lax.fori_loop(0, TSTEP, row_body, 0)

            fire_store(b, r)

            @pl.when(b + RING < BPW)
            def _():
                fire_gather(b + RING, r)

        return carry

    lax.fori_loop(0, BPW // RING, group, 0)

    for b in range((BPW // RING) * RING, BPW):
        r = b % RING
        wait_gather(b, r)
        wait_store(b - RING, r)

        def row_body(t, tc_):
            for cc in range(VPR):
                sl = pl.ds(16 * cc, LANES)
                obuf[r][t, sl] = gbuf[r][t, sl] * SCALE
            return tc_

        lax.fori_loop(0, TSTEP, row_body, 0)
        fire_store(b, r)

    for b in range(BPW - RING, BPW):
        wait_store(b, b % RING)


@jax.jit
def _emb_lookup(table, tokens):
    mesh = plsc.VectorSubcoreMesh(core_axis_name="c", subcore_axis_name="s")
    fn = pl.kernel(
        _emb_kernel_body,
        mesh=mesh,
        out_type=jax.ShapeDtypeStruct((BATCH, TSTEP, EMB), jnp.float32),
        scratch_types=[
            pltpu.VMEM((BPW, TSTEP), jnp.int32),
            pltpu.VMEM((TSTEP, EMB), jnp.float32),
            pltpu.VMEM((TSTEP, EMB), jnp.float32),
            pltpu.VMEM((TSTEP, EMB), jnp.float32),
            pltpu.VMEM((TSTEP, EMB), jnp.float32),
            pltpu.VMEM((TSTEP, EMB), jnp.float32),
            pltpu.VMEM((TSTEP, EMB), jnp.float32),
            pltpu.SemaphoreType.DMA((RING,)),
            pltpu.SemaphoreType.DMA((RING,)),
        ],
        compiler_params=pltpu.CompilerParams(use_tc_tiling_on_sc=False),
    )
    return fn(table, tokens)


def kernel(tokens, table):
    return _emb_lookup(table, tokens.astype(jnp.int32))
